# Initial kernel scaffold; baseline (speedup 1.0000x reference)
import jax
import jax.numpy as jnp
from jax.experimental import pallas as pl


def kernel(x, edge_index, W1, b1, g1, be1, W2, b2, g2, be2, W3, b3, g3, be3, L1w, L1b, L2w, L2b, g4, be4, Ow, Ob):
    raise NotImplementedError("write your pallas kernel here")


# speedup vs baseline: 3.1057x; 3.1057x over previous
import functools
import jax
import jax.numpy as jnp
from jax import lax
from jax.experimental import pallas as pl
from jax.experimental.pallas import tpu as pltpu, tpu_sc as plsc

N = 10000
E = 320000
D = 128
NT = 32
R = 320
NPAD = NT * R
EC = E // NT
EPAD = E + 1536
DUMMY = E + 1528
CH = 1024
NEG = -3.0e38

_MESH = dict(core_axis_name="c", subcore_axis_name="s", num_cores=2,
             num_subcores=16)

_i32 = jnp.int32
_f32 = jnp.float32


def _wid():
    return lax.axis_index("s") * 2 + lax.axis_index("c")


def _lane_iota():
    return lax.broadcasted_iota(_i32, (16,), 0)


def _bucket(d):
    return (d * 6554) >> 21


def _hist_body(edge_hbm, hist_hbm, dstv, histv):
    t = _wid()
    io = _lane_iota()
    pltpu.sync_copy(edge_hbm.at[1, pl.ds(t * EC, EC)], dstv.at[pl.ds(0, EC)])

    def zero16(i, _):
        histv[pl.ds(i * 16, 16)] = jnp.zeros((16,), _i32)
        return 0

    lax.fori_loop(0, 3, zero16, 0)

    def edge(e, _):
        d = dstv[pl.ds(e, 16)][0]
        b = _bucket(d)
        b16 = b & ~15
        w = histv[pl.ds(b16, 16)]
        histv[pl.ds(b16, 16)] = jnp.where(io == (b - b16), w + 1, w)
        return 0

    lax.fori_loop(0, EC, edge, 0)
    pltpu.sync_copy(histv.at[pl.ds(0, 32)], hist_hbm.at[pl.ds(t * 32, 32)])


def _place_body(edge_hbm, hist_hbm, rdst_hbm, rsrc_hbm, meta_hbm,
                dstv, srcv, histv, cursor, metav, slots2d):
    t = _wid()
    io = _lane_iota()
    pltpu.sync_copy(edge_hbm.at[0, pl.ds(t * EC, EC)], srcv.at[pl.ds(0, EC)])
    pltpu.sync_copy(edge_hbm.at[1, pl.ds(t * EC, EC)], dstv.at[pl.ds(0, EC)])
    pltpu.sync_copy(hist_hbm, histv.at[pl.ds(0, NT * 32)])

    def per_bucket(b, start):
        def per_tile(tp, carry):
            tot, part = carry
            h = histv[pl.ds(tp * 32 + b, 16)][0]
            part = jnp.where(tp < t, part + h, part)
            return (tot + h, part)

        tot, part = lax.fori_loop(0, NT, per_tile, (0, 0))
        b16 = b & ~15
        cw = cursor[pl.ds(b16, 16)]
        cursor[pl.ds(b16, 16)] = jnp.where(io == (b - b16), start + part, cw)
        mw = metav[pl.ds(b16, 16)]
        metav[pl.ds(b16, 16)] = jnp.where(io == (b - b16), start, mw)
        mw2 = metav[pl.ds(32 + b16, 16)]
        metav[pl.ds(32 + b16, 16)] = jnp.where(io == (b - b16), tot, mw2)
        return start + (((tot + 7) >> 3) << 3)

    lax.fori_loop(0, 32, per_bucket, 0)

    @pl.when(t == 0)
    def _():
        pltpu.sync_copy(metav.at[pl.ds(0, 64)], meta_hbm)

    def initrow(i, _):
        slots2d[i >> 3, pl.ds((i & 7) * 16, 16)] = jnp.full((16,), DUMMY, _i32)
        return 0

    lax.fori_loop(0, (EC + 240) // 16, initrow, 0)

    def edge(e, _):
        d = dstv[pl.ds(e, 16)][0]
        b = _bucket(d)
        b16 = b & ~15
        slot = cursor[pl.ds(b, 16)][0]
        cw = cursor[pl.ds(b16, 16)]
        cursor[pl.ds(b16, 16)] = jnp.where(io == (b - b16), cw + 1, cw)
        j = e >> 7
        l = e & 127
        l16 = l & ~15
        sw = slots2d[j, pl.ds(l16, 16)]
        slots2d[j, pl.ds(l16, 16)] = jnp.where(io == (l - l16), slot, sw)
        return 0

    lax.fori_loop(0, EC, edge, 0)

    def scat(j, _):
        pltpu.sync_copy(dstv.at[pl.ds(j * 128, 128)],
                        rdst_hbm.at[slots2d.at[j]])
        pltpu.sync_copy(srcv.at[pl.ds(j * 128, 128)],
                        rsrc_hbm.at[slots2d.at[j]])
        return 0

    lax.fori_loop(0, (EC + 240) // 128, scat, 0)


def _acc_body(rdst_hbm, rsrc_hbm, meta_hbm, xd_hbm, xs_hbm,
              ssum_hbm, smax_hbm,
              metav, dstg, srcg, dstw, rows_d, rows_s, acc_s, acc_m, sem):
    t = _wid()
    pltpu.sync_copy(meta_hbm, metav.at[pl.ds(0, 64)])
    my_start = metav[pl.ds(t, 16)][0]
    my_cnt = metav[pl.ds(32 + t, 16)][0]

    def initacc(r, _):
        acc_s[r, pl.ds(0, 16)] = jnp.zeros((16,), _f32)
        acc_m[r, pl.ds(0, 16)] = jnp.full((16,), NEG, _f32)
        return 0

    lax.fori_loop(0, R, initacc, 0)

    base_row = t * R

    def chunk(c, _):
        off = pl.multiple_of(my_start + c * CH, 8)
        pltpu.sync_copy(rdst_hbm.at[pl.ds(off, CH)], dstg)
        pltpu.sync_copy(rsrc_hbm.at[pl.ds(off, CH)], srcg)
        pltpu.sync_copy(rdst_hbm.at[pl.ds(off, CH)], dstw.at[pl.ds(0, CH)])

        def clamp(i, _):
            v = dstg[pl.ds(i * 16, 16)]
            dstg[pl.ds(i * 16, 16)] = jnp.minimum(jnp.maximum(v, 0), N - 1)
            w = srcg[pl.ds(i * 16, 16)]
            srcg[pl.ds(i * 16, 16)] = jnp.minimum(jnp.maximum(w, 0), N - 1)
            return 0

        lax.fori_loop(0, CH // 16, clamp, 0)

        handles = []
        for k in range(CH // 128):
            handles.append(pltpu.async_copy(
                xd_hbm.at[dstg.at[pl.ds(k * 128, 128)]],
                rows_d.at[pl.ds(k * 128, 128), :], sem))
            handles.append(pltpu.async_copy(
                xs_hbm.at[srcg.at[pl.ds(k * 128, 128)]],
                rows_s.at[pl.ds(k * 128, 128), :], sem))
        for h in handles:
            h.wait()

        me = jnp.minimum(CH, my_cnt - c * CH)

        def edge(i, _):
            v = rows_d[i] + rows_s[i]
            dl = dstw[pl.ds(i, 16)][0] - base_row
            acc_s[dl, pl.ds(0, 16)] = acc_s[dl, pl.ds(0, 16)] + v
            acc_m[dl, pl.ds(0, 16)] = jnp.maximum(acc_m[dl, pl.ds(0, 16)], v)
            return 0

        lax.fori_loop(0, me, edge, 0)
        return 0

    nchunks = (my_cnt + CH - 1) >> 10
    lax.fori_loop(0, nchunks, chunk, 0)

    pltpu.sync_copy(acc_s, ssum_hbm.at[pl.ds(base_row, R), :])
    pltpu.sync_copy(acc_m, smax_hbm.at[pl.ds(base_row, R), :])


def _lrelu(t):
    return jnp.where(t >= 0, t, 0.01 * t)


def _proj_body(x_ref, wd_ref, ws_ref, od_ref, os_ref, *, fnext):
    cc = 0.5 * (lax.broadcasted_iota(_i32, (1, 16), 1) == fnext).astype(_f32)
    x = x_ref[...]
    od_ref[...] = jax.lax.dot_general(
        x, wd_ref[...], (((1,), (0,)), ((), ())),
        preferred_element_type=_f32) + cc
    os_ref[...] = jax.lax.dot_general(
        x, ws_ref[...], (((1,), (0,)), ((), ())),
        preferred_element_type=_f32) + cc


def _bn_blocks(ssum, smax, bias, f, gs, bes):
    cnt = ssum[:, f:f + 1]
    s_sum = ssum + cnt * bias
    s_mean = s_sum / jnp.maximum(cnt, 1.0)
    s_max = jnp.where(cnt > 0, smax + bias, 0.0)
    outs = []
    for h, g, be in zip((s_sum, s_mean, s_max), gs, bes):
        mu = jnp.mean(h, axis=0, keepdims=True)
        var = jnp.mean((h - mu) ** 2, axis=0, keepdims=True)
        hn = (h - mu) * lax.rsqrt(var + 1e-5) * g + be
        outs.append(_lrelu(hn))
    return outs


def _mid_body(ssum_ref, smax_ref, b_ref, g1_ref, g2_ref, g3_ref,
              be1_ref, be2_ref, be3_ref,
              da_ref, db_ref, dc_ref, sa_ref, sb_ref, sc_ref,
              od_ref, os_ref, *, f, fnext):
    hs, hm, hx = _bn_blocks(ssum_ref[...], smax_ref[...], b_ref[...], f,
                            (g1_ref[...], g2_ref[...], g3_ref[...]),
                            (be1_ref[...], be2_ref[...], be3_ref[...]))
    cc = 0.5 * (lax.broadcasted_iota(_i32, (1, 16), 1) == fnext).astype(_f32)
    dn = (((1,), (0,)), ((), ()))
    od_ref[...] = (jax.lax.dot_general(hs, da_ref[...], dn, preferred_element_type=_f32)
                   + jax.lax.dot_general(hm, db_ref[...], dn, preferred_element_type=_f32)
                   + jax.lax.dot_general(hx, dc_ref[...], dn, preferred_element_type=_f32)
                   + cc)
    os_ref[...] = (jax.lax.dot_general(hs, sa_ref[...], dn, preferred_element_type=_f32)
                   + jax.lax.dot_general(hm, sb_ref[...], dn, preferred_element_type=_f32)
                   + jax.lax.dot_general(hx, sc_ref[...], dn, preferred_element_type=_f32)
                   + cc)


def _head_body(ssum_ref, smax_ref, b_ref, g1_ref, g2_ref, g3_ref,
               be1_ref, be2_ref, be3_ref,
               la_ref, lb_ref, lc_ref, l1b_ref, l2_ref, l2b_ref,
               g4_ref, be4_ref, ow_ref, ob_ref, out_ref, *, f):
    hs, hm, hx = _bn_blocks(ssum_ref[...], smax_ref[...], b_ref[...], f,
                            (g1_ref[...], g2_ref[...], g3_ref[...]),
                            (be1_ref[...], be2_ref[...], be3_ref[...]))
    dn = (((1,), (0,)), ((), ()))
    v = _lrelu(jax.lax.dot_general(hs, la_ref[...], dn, preferred_element_type=_f32)
               + jax.lax.dot_general(hm, lb_ref[...], dn, preferred_element_type=_f32)
               + jax.lax.dot_general(hx, lc_ref[...], dn, preferred_element_type=_f32)
               + l1b_ref[...])
    z = jax.lax.dot_general(v, l2_ref[...], dn, preferred_element_type=_f32) + l2b_ref[...]
    mu = jnp.mean(z, axis=0, keepdims=True)
    var = jnp.mean((z - mu) ** 2, axis=0, keepdims=True)
    z = _lrelu((z - mu) * lax.rsqrt(var + 1e-5) * g4_ref[...] + be4_ref[...])
    res = jax.lax.dot_general(z, ow_ref[...], dn, preferred_element_type=_f32) + ob_ref[...]
    out_ref[...] = res[:, 0:1]


def _pad2(a, rows, cols):
    return jnp.pad(a, ((0, rows - a.shape[0]), (0, cols - a.shape[1])))


def _pad1(a, n):
    return jnp.pad(a, (0, n - a.shape[0])).reshape(1, n)


def _tc_call(body, out_shapes, *args):
    return pl.pallas_call(
        body,
        out_shape=tuple(jax.ShapeDtypeStruct(s, _f32) for s in out_shapes),
    )(*args)


def _sc_hist(edge_index):
    return pl.kernel(
        _hist_body,
        out_type=jax.ShapeDtypeStruct((NT * 32,), _i32),
        mesh=plsc.VectorSubcoreMesh(**_MESH),
        compiler_params=pltpu.CompilerParams(use_tc_tiling_on_sc=False),
        scratch_types=[
            pltpu.VMEM((EC + 16,), _i32),
            pltpu.VMEM((48,), _i32),
        ],
    )(edge_index)


def _sc_place(edge_index, hist):
    return pl.kernel(
        _place_body,
        out_type=(jax.ShapeDtypeStruct((EPAD,), _i32),
                  jax.ShapeDtypeStruct((EPAD,), _i32),
                  jax.ShapeDtypeStruct((64,), _i32)),
        mesh=plsc.VectorSubcoreMesh(**_MESH),
        compiler_params=pltpu.CompilerParams(use_tc_tiling_on_sc=False),
        scratch_types=[
            pltpu.VMEM((EC + 240 + 16,), _i32),
            pltpu.VMEM((EC + 240 + 16,), _i32),
            pltpu.VMEM((NT * 32 + 16,), _i32),
            pltpu.VMEM((48,), _i32),
            pltpu.VMEM((80,), _i32),
            pltpu.VMEM(((EC + 240) // 128, 128), _i32),
        ],
    )(edge_index, hist)


def _sc_acc(rdst, rsrc, meta, xd, xs):
    return pl.kernel(
        _acc_body,
        out_type=(jax.ShapeDtypeStruct((NPAD, 16), _f32),
                  jax.ShapeDtypeStruct((NPAD, 16), _f32)),
        mesh=plsc.VectorSubcoreMesh(**_MESH),
        compiler_params=pltpu.CompilerParams(use_tc_tiling_on_sc=False),
        scratch_types=[
            pltpu.VMEM((80,), _i32),
            pltpu.VMEM((CH,), _i32),
            pltpu.VMEM((CH,), _i32),
            pltpu.VMEM((CH + 16,), _i32),
            pltpu.VMEM((CH, 16), _f32),
            pltpu.VMEM((CH, 16), _f32),
            pltpu.VMEM((R, 16), _f32),
            pltpu.VMEM((R, 16), _f32),
            pltpu.SemaphoreType.DMA,
        ],
    )(rdst, rsrc, meta, xd, xs)


def kernel(x, edge_index, W1, b1, g1, be1, W2, b2, g2, be2, W3, b3, g3, be3,
           L1w, L1b, L2w, L2b, g4, be4, Ow, Ob):
    f1, f2, f3 = 12, 9, 7

    wd1 = _pad2(W1[:, :D].T, D, 16)
    ws1 = _pad2(W1[:, D:].T, D, 16)
    b1p = _pad1(b1, 16)
    g1p = [_pad1(g1[i * f1:(i + 1) * f1], 16) for i in range(3)]
    be1p = [_pad1(be1[i * f1:(i + 1) * f1], 16) for i in range(3)]
    w2d = [_pad2(W2[:, i * f1:(i + 1) * f1].T, 16, 16) for i in range(3)]
    w2s = [_pad2(W2[:, 3 * f1 + i * f1:3 * f1 + (i + 1) * f1].T, 16, 16)
           for i in range(3)]
    b2p = _pad1(b2, 16)
    g2p = [_pad1(g2[i * f2:(i + 1) * f2], 16) for i in range(3)]
    be2p = [_pad1(be2[i * f2:(i + 1) * f2], 16) for i in range(3)]
    w3d = [_pad2(W3[:, i * f2:(i + 1) * f2].T, 16, 16) for i in range(3)]
    w3s = [_pad2(W3[:, 3 * f2 + i * f2:3 * f2 + (i + 1) * f2].T, 16, 16)
           for i in range(3)]
    b3p = _pad1(b3, 16)
    g3p = [_pad1(g3[i * f3:(i + 1) * f3], 16) for i in range(3)]
    be3p = [_pad1(be3[i * f3:(i + 1) * f3], 16) for i in range(3)]
    l1p = [_pad2(L1w[:, i * f3:(i + 1) * f3].T, 16, 32) for i in range(3)]
    l1bp = _pad1(L1b, 32)
    l2p = _pad2(L2w.T, 32, 16)
    l2bp = _pad1(L2b, 16)
    g4p = _pad1(g4, 16)
    be4p = _pad1(be4, 16)
    owp = _pad2(Ow.T, 16, 8)
    obp = _pad1(Ob, 8)

    hist = _sc_hist(edge_index)
    rdst, rsrc, meta = _sc_place(edge_index, hist)

    xd1, xs1 = _tc_call(functools.partial(_proj_body, fnext=f1),
                        (((N, 16)), ((N, 16))), x, wd1, ws1)
    ss1, sm1 = _sc_acc(rdst, rsrc, meta, xd1, xs1)

    xd2, xs2 = _tc_call(
        functools.partial(_mid_body, f=f1, fnext=f2),
        ((N, 16), (N, 16)),
        ss1[:N], sm1[:N], b1p, *g1p, *be1p, *w2d, *w2s)
    ss2, sm2 = _sc_acc(rdst, rsrc, meta, xd2, xs2)

    xd3, xs3 = _tc_call(
        functools.partial(_mid_body, f=f2, fnext=f3),
        ((N, 16), (N, 16)),
        ss2[:N], sm2[:N], b2p, *g2p, *be2p, *w3d, *w3s)
    ss3, sm3 = _sc_acc(rdst, rsrc, meta, xd3, xs3)

    (out,) = _tc_call(
        functools.partial(_head_body, f=f3),
        ((N, 1),),
        ss3[:N], sm3[:N], b3p, *g3p, *be3p,
        *l1p, l1bp, l2p, l2bp, g4p, be4p, owp, obp)
    return out


# speedup vs baseline: 3.4841x; 1.1218x over previous
import functools
import jax
import jax.numpy as jnp
from jax import lax
from jax.experimental import pallas as pl
from jax.experimental.pallas import tpu as pltpu, tpu_sc as plsc

N = 10000
E = 320000
D = 128
NT = 32
R = 320
NPAD = NT * R
EC = E // NT
EPAD = E + 1536
DUMMY = E + 1528
CH = 1024
NEG = -3.0e38

_MESH = dict(core_axis_name="c", subcore_axis_name="s", num_cores=2,
             num_subcores=16)

_i32 = jnp.int32
_f32 = jnp.float32


def _wid():
    return lax.axis_index("s") * 2 + lax.axis_index("c")


def _lane_iota():
    return lax.broadcasted_iota(_i32, (16,), 0)


def _bucket(d):
    return (d * 6554) >> 21


def _hist_body(edge_hbm, hist_hbm, dstv, histv, idxbuf):
    t = _wid()
    io = _lane_iota()
    pltpu.sync_copy(edge_hbm.at[1, pl.ds(t * EC, EC)], dstv.at[pl.ds(0, EC)])

    def zero16(i, _):
        histv[pl.ds(i * 16, 16)] = jnp.zeros((16,), _i32)
        return 0

    lax.fori_loop(0, 3, zero16, 0)

    def grp(g, _):
        d = dstv[pl.ds(g * 16, 16)]
        bs = plsc.sort_key_val(_bucket(d), io)[0]
        nxt = bs[jnp.minimum(io + 1, 15)]
        start = (bs != bs[jnp.maximum(io - 1, 0)]) | (io == 0)
        end = (bs != nxt) | (io == 15)
        rank = io - plsc.cummax(jnp.where(start, io, 0))
        plsc.addupdate_scatter(histv, [bs], rank + 1, mask=end)
        return 0

    lax.fori_loop(0, EC // 16, grp, 0)
    idxbuf[0, pl.ds(0, 16)] = io * 32 + t
    idxbuf[0, pl.ds(16, 16)] = (io + 16) * 32 + t
    pltpu.sync_copy(histv.at[pl.ds(0, 32)], hist_hbm.at[idxbuf.at[0]])


def _place_body(edge_hbm, hist_hbm, rdst_hbm, rsrc_hbm, meta_hbm,
                dstv, srcv, histv, cursor, metav, slots2d):
    t = _wid()
    io = _lane_iota()
    pltpu.sync_copy(edge_hbm.at[0, pl.ds(t * EC, EC)], srcv.at[pl.ds(0, EC)])
    pltpu.sync_copy(edge_hbm.at[1, pl.ds(t * EC, EC)], dstv.at[pl.ds(0, EC)])
    pltpu.sync_copy(hist_hbm, histv.at[pl.ds(0, NT * 32)])

    def per_bucket(b, start):
        col0 = histv[pl.ds(b * 32, 16)]
        col1 = histv[pl.ds(b * 32 + 16, 16)]
        tot = jnp.sum(col0) + jnp.sum(col1)
        part = (jnp.sum(jnp.where(io < t, col0, 0))
                + jnp.sum(jnp.where(io + 16 < t, col1, 0)))
        b16 = b & ~15
        cw = cursor[pl.ds(b16, 16)]
        cursor[pl.ds(b16, 16)] = jnp.where(io == (b - b16), start + part, cw)
        mw = metav[pl.ds(b16, 16)]
        metav[pl.ds(b16, 16)] = jnp.where(io == (b - b16), start, mw)
        mw2 = metav[pl.ds(32 + b16, 16)]
        metav[pl.ds(32 + b16, 16)] = jnp.where(io == (b - b16), tot, mw2)
        return start + (((tot + 7) >> 3) << 3)

    lax.fori_loop(0, 32, per_bucket, 0)

    @pl.when(t == 0)
    def _():
        pltpu.sync_copy(metav.at[pl.ds(0, 64)], meta_hbm)

    def initrow(i, _):
        slots2d[i >> 3, pl.ds((i & 7) * 16, 16)] = jnp.full((16,), DUMMY, _i32)
        return 0

    lax.fori_loop(0, (EC + 240) // 16, initrow, 0)

    def grp(g, _):
        d = dstv[pl.ds(g * 16, 16)]
        bs, ls = plsc.sort_key_val(_bucket(d), io)
        start = (bs != bs[jnp.maximum(io - 1, 0)]) | (io == 0)
        end = (bs != bs[jnp.minimum(io + 1, 15)]) | (io == 15)
        rank = io - plsc.cummax(jnp.where(start, io, 0))
        curs = plsc.load_gather(cursor, [bs])
        slots = curs + rank
        plsc.store_scatter(cursor, [bs], slots + 1, mask=end)
        plsc.store_scatter(slots2d, [jnp.full((16,), 0, _i32) + (g >> 3),
                                     (g & 7) * 16 + ls], slots)
        return 0

    lax.fori_loop(0, EC // 16, grp, 0)

    def scat(j, _):
        pltpu.sync_copy(dstv.at[pl.ds(j * 128, 128)],
                        rdst_hbm.at[slots2d.at[j]])
        pltpu.sync_copy(srcv.at[pl.ds(j * 128, 128)],
                        rsrc_hbm.at[slots2d.at[j]])
        return 0

    lax.fori_loop(0, (EC + 240) // 128, scat, 0)


def _acc_body(rdst_hbm, rsrc_hbm, meta_hbm, xd_hbm, xs_hbm,
              ssum_hbm, smax_hbm,
              metav, dstg, srcg, dstw, rows_d, rows_s,
              acc_s, acc_m, acc_s1, acc_m1, acc_s2, acc_m2,
              acc_s3, acc_m3, sem):
    t = _wid()
    pltpu.sync_copy(meta_hbm, metav.at[pl.ds(0, 64)])
    my_start = metav[pl.ds(t, 16)][0]
    my_cnt = metav[pl.ds(32 + t, 16)][0]

    def initacc(r, _):
        for s_ref, m_ref in ((acc_s, acc_m), (acc_s1, acc_m1),
                             (acc_s2, acc_m2), (acc_s3, acc_m3)):
            s_ref[r, pl.ds(0, 16)] = jnp.zeros((16,), _f32)
            m_ref[r, pl.ds(0, 16)] = jnp.full((16,), NEG, _f32)
        return 0

    lax.fori_loop(0, R, initacc, 0)

    base_row = t * R

    def chunk(c, _):
        off = pl.multiple_of(my_start + c * CH, 8)
        pltpu.sync_copy(rdst_hbm.at[pl.ds(off, CH)], dstg)
        pltpu.sync_copy(rsrc_hbm.at[pl.ds(off, CH)], srcg)
        pltpu.sync_copy(rdst_hbm.at[pl.ds(off, CH)], dstw.at[pl.ds(0, CH)])

        def clamp(i, _):
            v = dstg[pl.ds(i * 16, 16)]
            dstg[pl.ds(i * 16, 16)] = jnp.minimum(jnp.maximum(v, 0), N - 1)
            w = srcg[pl.ds(i * 16, 16)]
            srcg[pl.ds(i * 16, 16)] = jnp.minimum(jnp.maximum(w, 0), N - 1)
            return 0

        lax.fori_loop(0, CH // 16, clamp, 0)

        handles = []
        for k in range(CH // 128):
            handles.append(pltpu.async_copy(
                xd_hbm.at[dstg.at[pl.ds(k * 128, 128)]],
                rows_d.at[pl.ds(k * 128, 128), :], sem))
            handles.append(pltpu.async_copy(
                xs_hbm.at[srcg.at[pl.ds(k * 128, 128)]],
                rows_s.at[pl.ds(k * 128, 128), :], sem))
        for h in handles:
            h.wait()

        me = jnp.minimum(CH, my_cnt - c * CH)
        q = me >> 2

        def upd(s_ref, m_ref, e):
            v = rows_d[e] + rows_s[e]
            dl = dstw[pl.ds(e, 16)][0] - base_row
            s_ref[dl, pl.ds(0, 16)] = s_ref[dl, pl.ds(0, 16)] + v
            m_ref[dl, pl.ds(0, 16)] = jnp.maximum(m_ref[dl, pl.ds(0, 16)], v)

        def edge4(i, _):
            upd(acc_s, acc_m, i)
            upd(acc_s1, acc_m1, q + i)
            upd(acc_s2, acc_m2, 2 * q + i)
            upd(acc_s3, acc_m3, 3 * q + i)
            return 0

        lax.fori_loop(0, q, edge4, 0)

        def tail(i, _):
            upd(acc_s, acc_m, i)
            return 0

        lax.fori_loop(4 * q, me, tail, 0)
        return 0

    nchunks = (my_cnt + CH - 1) >> 10
    lax.fori_loop(0, nchunks, chunk, 0)

    def mrg(r, _):
        rs = pl.ds(0, 16)
        acc_s[r, rs] = (acc_s[r, rs] + acc_s1[r, rs]) + (acc_s2[r, rs] + acc_s3[r, rs])
        acc_m[r, rs] = jnp.maximum(jnp.maximum(acc_m[r, rs], acc_m1[r, rs]),
                                   jnp.maximum(acc_m2[r, rs], acc_m3[r, rs]))
        return 0

    lax.fori_loop(0, R, mrg, 0)

    pltpu.sync_copy(acc_s, ssum_hbm.at[pl.ds(base_row, R), :])
    pltpu.sync_copy(acc_m, smax_hbm.at[pl.ds(base_row, R), :])


def _lrelu(t):
    return jnp.where(t >= 0, t, 0.01 * t)


def _proj_body(x_ref, wd_ref, ws_ref, od_ref, os_ref, *, fnext):
    cc = 0.5 * (lax.broadcasted_iota(_i32, (1, 16), 1) == fnext).astype(_f32)
    x = x_ref[...]
    od_ref[...] = jax.lax.dot_general(
        x, wd_ref[...], (((1,), (0,)), ((), ())),
        preferred_element_type=_f32) + cc
    os_ref[...] = jax.lax.dot_general(
        x, ws_ref[...], (((1,), (0,)), ((), ())),
        preferred_element_type=_f32) + cc


def _bn_blocks(ssum, smax, bias, f, gs, bes):
    cnt = ssum[:, f:f + 1]
    s_sum = ssum + cnt * bias
    s_mean = s_sum / jnp.maximum(cnt, 1.0)
    s_max = jnp.where(cnt > 0, smax + bias, 0.0)
    outs = []
    for h, g, be in zip((s_sum, s_mean, s_max), gs, bes):
        mu = jnp.mean(h, axis=0, keepdims=True)
        var = jnp.mean((h - mu) ** 2, axis=0, keepdims=True)
        hn = (h - mu) * lax.rsqrt(var + 1e-5) * g + be
        outs.append(_lrelu(hn))
    return outs


def _mid_body(ssum_ref, smax_ref, b_ref, g1_ref, g2_ref, g3_ref,
              be1_ref, be2_ref, be3_ref,
              da_ref, db_ref, dc_ref, sa_ref, sb_ref, sc_ref,
              od_ref, os_ref, *, f, fnext):
    hs, hm, hx = _bn_blocks(ssum_ref[...], smax_ref[...], b_ref[...], f,
                            (g1_ref[...], g2_ref[...], g3_ref[...]),
                            (be1_ref[...], be2_ref[...], be3_ref[...]))
    cc = 0.5 * (lax.broadcasted_iota(_i32, (1, 16), 1) == fnext).astype(_f32)
    dn = (((1,), (0,)), ((), ()))
    od_ref[...] = (jax.lax.dot_general(hs, da_ref[...], dn, preferred_element_type=_f32)
                   + jax.lax.dot_general(hm, db_ref[...], dn, preferred_element_type=_f32)
                   + jax.lax.dot_general(hx, dc_ref[...], dn, preferred_element_type=_f32)
                   + cc)
    os_ref[...] = (jax.lax.dot_general(hs, sa_ref[...], dn, preferred_element_type=_f32)
                   + jax.lax.dot_general(hm, sb_ref[...], dn, preferred_element_type=_f32)
                   + jax.lax.dot_general(hx, sc_ref[...], dn, preferred_element_type=_f32)
                   + cc)


def _head_body(ssum_ref, smax_ref, b_ref, g1_ref, g2_ref, g3_ref,
               be1_ref, be2_ref, be3_ref,
               la_ref, lb_ref, lc_ref, l1b_ref, l2_ref, l2b_ref,
               g4_ref, be4_ref, ow_ref, ob_ref, out_ref, *, f):
    hs, hm, hx = _bn_blocks(ssum_ref[...], smax_ref[...], b_ref[...], f,
                            (g1_ref[...], g2_ref[...], g3_ref[...]),
                            (be1_ref[...], be2_ref[...], be3_ref[...]))
    dn = (((1,), (0,)), ((), ()))
    v = _lrelu(jax.lax.dot_general(hs, la_ref[...], dn, preferred_element_type=_f32)
               + jax.lax.dot_general(hm, lb_ref[...], dn, preferred_element_type=_f32)
               + jax.lax.dot_general(hx, lc_ref[...], dn, preferred_element_type=_f32)
               + l1b_ref[...])
    z = jax.lax.dot_general(v, l2_ref[...], dn, preferred_element_type=_f32) + l2b_ref[...]
    mu = jnp.mean(z, axis=0, keepdims=True)
    var = jnp.mean((z - mu) ** 2, axis=0, keepdims=True)
    z = _lrelu((z - mu) * lax.rsqrt(var + 1e-5) * g4_ref[...] + be4_ref[...])
    res = jax.lax.dot_general(z, ow_ref[...], dn, preferred_element_type=_f32) + ob_ref[...]
    out_ref[...] = res[:, 0:1]


def _pad2(a, rows, cols):
    return jnp.pad(a, ((0, rows - a.shape[0]), (0, cols - a.shape[1])))


def _pad1(a, n):
    return jnp.pad(a, (0, n - a.shape[0])).reshape(1, n)


def _tc_call(body, out_shapes, *args):
    return pl.pallas_call(
        body,
        out_shape=tuple(jax.ShapeDtypeStruct(s, _f32) for s in out_shapes),
    )(*args)


def _sc_hist(edge_index):
    return pl.kernel(
        _hist_body,
        out_type=jax.ShapeDtypeStruct((NT * 32,), _i32),
        mesh=plsc.VectorSubcoreMesh(**_MESH),
        compiler_params=pltpu.CompilerParams(use_tc_tiling_on_sc=False,
                                             needs_layout_passes=False),
        scratch_types=[
            pltpu.VMEM((EC + 16,), _i32),
            pltpu.VMEM((1040,), _i32),
            pltpu.VMEM((1, 32), _i32),
        ],
    )(edge_index)


def _sc_place(edge_index, hist):
    return pl.kernel(
        _place_body,
        out_type=(jax.ShapeDtypeStruct((EPAD,), _i32),
                  jax.ShapeDtypeStruct((EPAD,), _i32),
                  jax.ShapeDtypeStruct((64,), _i32)),
        mesh=plsc.VectorSubcoreMesh(**_MESH),
        compiler_params=pltpu.CompilerParams(use_tc_tiling_on_sc=False,
                                             needs_layout_passes=False),
        scratch_types=[
            pltpu.VMEM((EC + 240 + 16,), _i32),
            pltpu.VMEM((EC + 240 + 16,), _i32),
            pltpu.VMEM((NT * 32 + 16,), _i32),
            pltpu.VMEM((48,), _i32),
            pltpu.VMEM((80,), _i32),
            pltpu.VMEM(((EC + 240) // 128, 128), _i32),
        ],
    )(edge_index, hist)


def _sc_acc(rdst, rsrc, meta, xd, xs):
    return pl.kernel(
        _acc_body,
        out_type=(jax.ShapeDtypeStruct((NPAD, 16), _f32),
                  jax.ShapeDtypeStruct((NPAD, 16), _f32)),
        mesh=plsc.VectorSubcoreMesh(**_MESH),
        compiler_params=pltpu.CompilerParams(use_tc_tiling_on_sc=False),
        scratch_types=[
            pltpu.VMEM((80,), _i32),
            pltpu.VMEM((CH,), _i32),
            pltpu.VMEM((CH,), _i32),
            pltpu.VMEM((CH + 16,), _i32),
            pltpu.VMEM((CH, 16), _f32),
            pltpu.VMEM((CH, 16), _f32),
            pltpu.VMEM((R, 16), _f32),
            pltpu.VMEM((R, 16), _f32),
            pltpu.VMEM((R, 16), _f32),
            pltpu.VMEM((R, 16), _f32),
            pltpu.VMEM((R, 16), _f32),
            pltpu.VMEM((R, 16), _f32),
            pltpu.VMEM((R, 16), _f32),
            pltpu.VMEM((R, 16), _f32),
            pltpu.SemaphoreType.DMA,
        ],
    )(rdst, rsrc, meta, xd, xs)


def kernel(x, edge_index, W1, b1, g1, be1, W2, b2, g2, be2, W3, b3, g3, be3,
           L1w, L1b, L2w, L2b, g4, be4, Ow, Ob):
    f1, f2, f3 = 12, 9, 7

    wd1 = _pad2(W1[:, :D].T, D, 16)
    ws1 = _pad2(W1[:, D:].T, D, 16)
    b1p = _pad1(b1, 16)
    g1p = [_pad1(g1[i * f1:(i + 1) * f1], 16) for i in range(3)]
    be1p = [_pad1(be1[i * f1:(i + 1) * f1], 16) for i in range(3)]
    w2d = [_pad2(W2[:, i * f1:(i + 1) * f1].T, 16, 16) for i in range(3)]
    w2s = [_pad2(W2[:, 3 * f1 + i * f1:3 * f1 + (i + 1) * f1].T, 16, 16)
           for i in range(3)]
    b2p = _pad1(b2, 16)
    g2p = [_pad1(g2[i * f2:(i + 1) * f2], 16) for i in range(3)]
    be2p = [_pad1(be2[i * f2:(i + 1) * f2], 16) for i in range(3)]
    w3d = [_pad2(W3[:, i * f2:(i + 1) * f2].T, 16, 16) for i in range(3)]
    w3s = [_pad2(W3[:, 3 * f2 + i * f2:3 * f2 + (i + 1) * f2].T, 16, 16)
           for i in range(3)]
    b3p = _pad1(b3, 16)
    g3p = [_pad1(g3[i * f3:(i + 1) * f3], 16) for i in range(3)]
    be3p = [_pad1(be3[i * f3:(i + 1) * f3], 16) for i in range(3)]
    l1p = [_pad2(L1w[:, i * f3:(i + 1) * f3].T, 16, 32) for i in range(3)]
    l1bp = _pad1(L1b, 32)
    l2p = _pad2(L2w.T, 32, 16)
    l2bp = _pad1(L2b, 16)
    g4p = _pad1(g4, 16)
    be4p = _pad1(be4, 16)
    owp = _pad2(Ow.T, 16, 8)
    obp = _pad1(Ob, 8)

    hist = _sc_hist(edge_index)
    rdst, rsrc, meta = _sc_place(edge_index, hist)

    xd1, xs1 = _tc_call(functools.partial(_proj_body, fnext=f1),
                        (((N, 16)), ((N, 16))), x, wd1, ws1)
    ss1, sm1 = _sc_acc(rdst, rsrc, meta, xd1, xs1)

    xd2, xs2 = _tc_call(
        functools.partial(_mid_body, f=f1, fnext=f2),
        ((N, 16), (N, 16)),
        ss1[:N], sm1[:N], b1p, *g1p, *be1p, *w2d, *w2s)
    ss2, sm2 = _sc_acc(rdst, rsrc, meta, xd2, xs2)

    xd3, xs3 = _tc_call(
        functools.partial(_mid_body, f=f2, fnext=f3),
        ((N, 16), (N, 16)),
        ss2[:N], sm2[:N], b2p, *g2p, *be2p, *w3d, *w3s)
    ss3, sm3 = _sc_acc(rdst, rsrc, meta, xd3, xs3)

    (out,) = _tc_call(
        functools.partial(_head_body, f=f3),
        ((N, 1),),
        ss3[:N], sm3[:N], b3p, *g3p, *be3p,
        *l1p, l1bp, l2p, l2bp, g4p, be4p, owp, obp)
    return out


# speedup vs baseline: 4.0231x; 1.1547x over previous
import functools
import jax
import jax.numpy as jnp
from jax import lax
from jax.experimental import pallas as pl
from jax.experimental.pallas import tpu as pltpu, tpu_sc as plsc

N = 10000
E = 320000
D = 128
NT = 32
R = 320
NPAD = NT * R
EC = E // NT
EPAD = E + 1536
DUMMY = E + 1528
CH = 1024
NEG = -3.0e38

_MESH = dict(core_axis_name="c", subcore_axis_name="s", num_cores=2,
             num_subcores=16)

_i32 = jnp.int32
_f32 = jnp.float32


def _wid():
    return lax.axis_index("s") * 2 + lax.axis_index("c")


def _lane_iota():
    return lax.broadcasted_iota(_i32, (16,), 0)


def _bucket(d):
    return (d * 6554) >> 21


def _hist_body(edge_hbm, hist_hbm, dstv, histv, idxbuf):
    t = _wid()
    io = _lane_iota()
    pltpu.sync_copy(edge_hbm.at[1, pl.ds(t * EC, EC)], dstv.at[pl.ds(0, EC)])

    def zero16(i, _):
        histv[pl.ds(i * 16, 16)] = jnp.zeros((16,), _i32)
        return 0

    lax.fori_loop(0, 3, zero16, 0)

    def grp(g, _):
        d = dstv[pl.ds(g * 16, 16)]
        bs = plsc.sort_key_val(_bucket(d), io)[0]
        nxt = bs[jnp.minimum(io + 1, 15)]
        start = (bs != bs[jnp.maximum(io - 1, 0)]) | (io == 0)
        end = (bs != nxt) | (io == 15)
        rank = io - plsc.cummax(jnp.where(start, io, 0))
        plsc.addupdate_scatter(histv, [bs], rank + 1, mask=end)
        return 0

    lax.fori_loop(0, EC // 16, grp, 0)
    idxbuf[0, pl.ds(0, 16)] = io * 32 + t
    idxbuf[0, pl.ds(16, 16)] = (io + 16) * 32 + t
    pltpu.sync_copy(histv.at[pl.ds(0, 32)], hist_hbm.at[idxbuf.at[0]])


def _place_body(edge_hbm, hist_hbm, rdst_hbm, rsrc_hbm, meta_hbm,
                dstv, srcv, histv, cursor, metav, slots2d, sem):
    t = _wid()
    io = _lane_iota()
    pltpu.sync_copy(edge_hbm.at[0, pl.ds(t * EC, EC)], srcv.at[pl.ds(0, EC)])
    pltpu.sync_copy(edge_hbm.at[1, pl.ds(t * EC, EC)], dstv.at[pl.ds(0, EC)])
    pltpu.sync_copy(hist_hbm, histv.at[pl.ds(0, NT * 32)])

    def per_bucket(b, start):
        col0 = histv[pl.ds(b * 32, 16)]
        col1 = histv[pl.ds(b * 32 + 16, 16)]
        tot = jnp.sum(col0) + jnp.sum(col1)
        part = (jnp.sum(jnp.where(io < t, col0, 0))
                + jnp.sum(jnp.where(io + 16 < t, col1, 0)))
        b16 = b & ~15
        cw = cursor[pl.ds(b16, 16)]
        cursor[pl.ds(b16, 16)] = jnp.where(io == (b - b16), start + part, cw)
        mw = metav[pl.ds(b * 8, 16)]
        metav[pl.ds(b * 8, 16)] = jnp.where(io == 0, start, mw)
        mw2 = metav[pl.ds(256 + b * 8, 16)]
        metav[pl.ds(256 + b * 8, 16)] = jnp.where(io == 0, tot, mw2)
        return start + (((tot + 7) >> 3) << 3)

    lax.fori_loop(0, 32, per_bucket, 0)

    @pl.when(t == 0)
    def _():
        pltpu.sync_copy(metav.at[pl.ds(0, 512)], meta_hbm)

    def initrow(i, _):
        slots2d[i >> 3, pl.ds((i & 7) * 16, 16)] = jnp.full((16,), DUMMY, _i32)
        return 0

    lax.fori_loop(0, (EC + 240) // 16, initrow, 0)

    def grp(g, _):
        d = dstv[pl.ds(g * 16, 16)]
        s = srcv[pl.ds(g * 16, 16)]
        bs, ls = plsc.sort_key_val(_bucket(d), io)
        start = (bs != bs[jnp.maximum(io - 1, 0)]) | (io == 0)
        end = (bs != bs[jnp.minimum(io + 1, 15)]) | (io == 15)
        rank = io - plsc.cummax(jnp.where(start, io, 0))
        curs = plsc.load_gather(cursor, [bs])
        slots = curs + rank
        plsc.store_scatter(cursor, [bs], slots + 1, mask=end)
        row = jnp.full((16,), 0, _i32) + (g >> 3)
        col = (g & 7) * 16 + ls
        plsc.store_scatter(slots2d, [row, col], slots)
        return 0

    lax.fori_loop(0, EC // 16, grp, 0)

    NJ = (EC + 240) // 128

    def scat8(blk, _):
        hs = []
        for u in range(4):
            j = blk * 4 + u
            hs.append(pltpu.async_copy(
                dstv.at[pl.ds(j * 128, 128)], rdst_hbm.at[slots2d.at[j]], sem))
            hs.append(pltpu.async_copy(
                srcv.at[pl.ds(j * 128, 128)], rsrc_hbm.at[slots2d.at[j]], sem))
        for h in hs:
            h.wait()
        return 0

    lax.fori_loop(0, NJ // 4, scat8, 0)


def _acc_body(rdst_hbm, rsrc_hbm, meta_hbm, xd_hbm, xs_hbm,
              ssum_hbm, smax_hbm,
              metav, dstg, srcg, dloc, rows_d, rows_s,
              acc_s, acc_m, acc_s1, acc_m1, acc_s2, acc_m2,
              acc_s3, acc_m3, sem):
    t = _wid()
    io = _lane_iota()
    pltpu.sync_copy(meta_hbm, metav.at[pl.ds(0, 512)])
    my_start = metav[pl.ds(t * 8, 16)][0]
    my_cnt = metav[pl.ds(256 + t * 8, 16)][0]

    def initacc(r, _):
        for s_ref, m_ref in ((acc_s, acc_m), (acc_s1, acc_m1),
                             (acc_s2, acc_m2), (acc_s3, acc_m3)):
            s_ref[r, pl.ds(0, 16)] = jnp.zeros((16,), _f32)
            m_ref[r, pl.ds(0, 16)] = jnp.full((16,), NEG, _f32)
        return 0

    lax.fori_loop(0, R + 8, initacc, 0)

    base_row = t * R
    sbank = (acc_s, acc_s1, acc_s2, acc_s3)
    mbank = (acc_m, acc_m1, acc_m2, acc_m3)

    def chunk(c, _):
        off = pl.multiple_of(my_start + c * CH, 8)
        pltpu.sync_copy(rdst_hbm.at[pl.ds(off, CH)], dstg)
        pltpu.sync_copy(rsrc_hbm.at[pl.ds(off, CH)], srcg)
        me = jnp.minimum(CH, my_cnt - c * CH)

        def mkidx(k, _):
            dg = dstg[pl.ds(k * 16, 16)]
            dg = jnp.minimum(jnp.maximum(dg, 0), N - 1)
            dstg[pl.ds(k * 16, 16)] = dg
            sg = srcg[pl.ds(k * 16, 16)]
            srcg[pl.ds(k * 16, 16)] = jnp.minimum(jnp.maximum(sg, 0), N - 1)
            dl = jnp.minimum(jnp.maximum(dg - base_row, 0), R)
            dloc[pl.ds(k * 16, 16)] = jnp.where(k * 16 + io < me, dl, R)
            return 0

        lax.fori_loop(0, CH // 16, mkidx, 0)

        handles = []
        for k in range(CH // 128):
            handles.append(pltpu.async_copy(
                xd_hbm.at[dstg.at[pl.ds(k * 128, 128)]],
                rows_d.at[pl.ds(k * 128, 128), :], sem))
            handles.append(pltpu.async_copy(
                xs_hbm.at[srcg.at[pl.ds(k * 128, 128)]],
                rows_s.at[pl.ds(k * 128, 128), :], sem))
        for h in handles:
            h.wait()

        def grp16(g, _):
            dlv = dloc[pl.ds(g * 16, 16)]
            for k in range(16):
                e = g * 16 + k
                dl = dlv[k]
                v = rows_d[e] + rows_s[e]
                s_ref = sbank[k & 3]
                m_ref = mbank[k & 3]
                s_ref[dl, pl.ds(0, 16)] = s_ref[dl, pl.ds(0, 16)] + v
                m_ref[dl, pl.ds(0, 16)] = jnp.maximum(m_ref[dl, pl.ds(0, 16)], v)
            return 0

        lax.fori_loop(0, (me + 15) >> 4, grp16, 0)
        return 0

    nchunks = (my_cnt + CH - 1) >> 10
    lax.fori_loop(0, nchunks, chunk, 0)

    def mrg(r, _):
        rs = pl.ds(0, 16)
        acc_s[r, rs] = (acc_s[r, rs] + acc_s1[r, rs]) + (acc_s2[r, rs] + acc_s3[r, rs])
        acc_m[r, rs] = jnp.maximum(jnp.maximum(acc_m[r, rs], acc_m1[r, rs]),
                                   jnp.maximum(acc_m2[r, rs], acc_m3[r, rs]))
        return 0

    lax.fori_loop(0, R, mrg, 0)

    pltpu.sync_copy(acc_s.at[pl.ds(0, R), :], ssum_hbm.at[pl.ds(base_row, R), :])
    pltpu.sync_copy(acc_m.at[pl.ds(0, R), :], smax_hbm.at[pl.ds(base_row, R), :])


def _lrelu(t):
    return jnp.where(t >= 0, t, 0.01 * t)


def _proj_body(x_ref, wd_ref, ws_ref, od_ref, os_ref, *, fnext):
    cc = 0.5 * (lax.broadcasted_iota(_i32, (1, 16), 1) == fnext).astype(_f32)
    x = x_ref[...]
    od_ref[...] = jax.lax.dot_general(
        x, wd_ref[...], (((1,), (0,)), ((), ())),
        preferred_element_type=_f32) + cc
    os_ref[...] = jax.lax.dot_general(
        x, ws_ref[...], (((1,), (0,)), ((), ())),
        preferred_element_type=_f32) + cc


def _bn_blocks(ssum, smax, bias, f, gs, bes):
    cnt = ssum[:, f:f + 1]
    s_sum = ssum + cnt * bias
    s_mean = s_sum / jnp.maximum(cnt, 1.0)
    s_max = jnp.where(cnt > 0, smax + bias, 0.0)
    outs = []
    for h, g, be in zip((s_sum, s_mean, s_max), gs, bes):
        mu = jnp.mean(h, axis=0, keepdims=True)
        var = jnp.mean((h - mu) ** 2, axis=0, keepdims=True)
        hn = (h - mu) * lax.rsqrt(var + 1e-5) * g + be
        outs.append(_lrelu(hn))
    return outs


def _mid_body(ssum_ref, smax_ref, b_ref, g1_ref, g2_ref, g3_ref,
              be1_ref, be2_ref, be3_ref,
              da_ref, db_ref, dc_ref, sa_ref, sb_ref, sc_ref,
              od_ref, os_ref, *, f, fnext):
    hs, hm, hx = _bn_blocks(ssum_ref[...], smax_ref[...], b_ref[...], f,
                            (g1_ref[...], g2_ref[...], g3_ref[...]),
                            (be1_ref[...], be2_ref[...], be3_ref[...]))
    cc = 0.5 * (lax.broadcasted_iota(_i32, (1, 16), 1) == fnext).astype(_f32)
    dn = (((1,), (0,)), ((), ()))
    od_ref[...] = (jax.lax.dot_general(hs, da_ref[...], dn, preferred_element_type=_f32)
                   + jax.lax.dot_general(hm, db_ref[...], dn, preferred_element_type=_f32)
                   + jax.lax.dot_general(hx, dc_ref[...], dn, preferred_element_type=_f32)
                   + cc)
    os_ref[...] = (jax.lax.dot_general(hs, sa_ref[...], dn, preferred_element_type=_f32)
                   + jax.lax.dot_general(hm, sb_ref[...], dn, preferred_element_type=_f32)
                   + jax.lax.dot_general(hx, sc_ref[...], dn, preferred_element_type=_f32)
                   + cc)


def _head_body(ssum_ref, smax_ref, b_ref, g1_ref, g2_ref, g3_ref,
               be1_ref, be2_ref, be3_ref,
               la_ref, lb_ref, lc_ref, l1b_ref, l2_ref, l2b_ref,
               g4_ref, be4_ref, ow_ref, ob_ref, out_ref, *, f):
    hs, hm, hx = _bn_blocks(ssum_ref[...], smax_ref[...], b_ref[...], f,
                            (g1_ref[...], g2_ref[...], g3_ref[...]),
                            (be1_ref[...], be2_ref[...], be3_ref[...]))
    dn = (((1,), (0,)), ((), ()))
    v = _lrelu(jax.lax.dot_general(hs, la_ref[...], dn, preferred_element_type=_f32)
               + jax.lax.dot_general(hm, lb_ref[...], dn, preferred_element_type=_f32)
               + jax.lax.dot_general(hx, lc_ref[...], dn, preferred_element_type=_f32)
               + l1b_ref[...])
    z = jax.lax.dot_general(v, l2_ref[...], dn, preferred_element_type=_f32) + l2b_ref[...]
    mu = jnp.mean(z, axis=0, keepdims=True)
    var = jnp.mean((z - mu) ** 2, axis=0, keepdims=True)
    z = _lrelu((z - mu) * lax.rsqrt(var + 1e-5) * g4_ref[...] + be4_ref[...])
    res = jax.lax.dot_general(z, ow_ref[...], dn, preferred_element_type=_f32) + ob_ref[...]
    out_ref[...] = res[:, 0:1]


def _pad2(a, rows, cols):
    return jnp.pad(a, ((0, rows - a.shape[0]), (0, cols - a.shape[1])))


def _pad1(a, n):
    return jnp.pad(a, (0, n - a.shape[0])).reshape(1, n)


def _tc_call(body, out_shapes, *args):
    return pl.pallas_call(
        body,
        out_shape=tuple(jax.ShapeDtypeStruct(s, _f32) for s in out_shapes),
    )(*args)


def _sc_hist(edge_index):
    return pl.kernel(
        _hist_body,
        out_type=jax.ShapeDtypeStruct((NT * 32,), _i32),
        mesh=plsc.VectorSubcoreMesh(**_MESH),
        compiler_params=pltpu.CompilerParams(use_tc_tiling_on_sc=False,
                                             needs_layout_passes=False),
        scratch_types=[
            pltpu.VMEM((EC + 16,), _i32),
            pltpu.VMEM((1040,), _i32),
            pltpu.VMEM((1, 32), _i32),
        ],
    )(edge_index)


def _sc_place(edge_index, hist):
    return pl.kernel(
        _place_body,
        out_type=(jax.ShapeDtypeStruct((EPAD,), _i32),
                  jax.ShapeDtypeStruct((EPAD,), _i32),
                  jax.ShapeDtypeStruct((512,), _i32)),
        mesh=plsc.VectorSubcoreMesh(**_MESH),
        compiler_params=pltpu.CompilerParams(use_tc_tiling_on_sc=False,
                                             needs_layout_passes=False),
        scratch_types=[
            pltpu.VMEM((EC + 240,), _i32),
            pltpu.VMEM((EC + 240,), _i32),
            pltpu.VMEM((NT * 32 + 16,), _i32),
            pltpu.VMEM((48,), _i32),
            pltpu.VMEM((528,), _i32),
            pltpu.VMEM(((EC + 240) // 128, 128), _i32),
            pltpu.SemaphoreType.DMA,
        ],
    )(edge_index, hist)


def _sc_acc(rdst, rsrc, meta, xd, xs):
    return pl.kernel(
        _acc_body,
        out_type=(jax.ShapeDtypeStruct((NPAD, 16), _f32),
                  jax.ShapeDtypeStruct((NPAD, 16), _f32)),
        mesh=plsc.VectorSubcoreMesh(**_MESH),
        compiler_params=pltpu.CompilerParams(use_tc_tiling_on_sc=False),
        scratch_types=[
            pltpu.VMEM((528,), _i32),
            pltpu.VMEM((CH,), _i32),
            pltpu.VMEM((CH,), _i32),
            pltpu.VMEM((CH,), _i32),
            pltpu.VMEM((CH, 16), _f32),
            pltpu.VMEM((CH, 16), _f32),
            pltpu.VMEM((R + 8, 16), _f32),
            pltpu.VMEM((R + 8, 16), _f32),
            pltpu.VMEM((R + 8, 16), _f32),
            pltpu.VMEM((R + 8, 16), _f32),
            pltpu.VMEM((R + 8, 16), _f32),
            pltpu.VMEM((R + 8, 16), _f32),
            pltpu.VMEM((R + 8, 16), _f32),
            pltpu.VMEM((R + 8, 16), _f32),
            pltpu.SemaphoreType.DMA,
        ],
    )(rdst, rsrc, meta, xd, xs)


def kernel(x, edge_index, W1, b1, g1, be1, W2, b2, g2, be2, W3, b3, g3, be3,
           L1w, L1b, L2w, L2b, g4, be4, Ow, Ob):
    f1, f2, f3 = 12, 9, 7

    wd1 = _pad2(W1[:, :D].T, D, 16)
    ws1 = _pad2(W1[:, D:].T, D, 16)
    b1p = _pad1(b1, 16)
    g1p = [_pad1(g1[i * f1:(i + 1) * f1], 16) for i in range(3)]
    be1p = [_pad1(be1[i * f1:(i + 1) * f1], 16) for i in range(3)]
    w2d = [_pad2(W2[:, i * f1:(i + 1) * f1].T, 16, 16) for i in range(3)]
    w2s = [_pad2(W2[:, 3 * f1 + i * f1:3 * f1 + (i + 1) * f1].T, 16, 16)
           for i in range(3)]
    b2p = _pad1(b2, 16)
    g2p = [_pad1(g2[i * f2:(i + 1) * f2], 16) for i in range(3)]
    be2p = [_pad1(be2[i * f2:(i + 1) * f2], 16) for i in range(3)]
    w3d = [_pad2(W3[:, i * f2:(i + 1) * f2].T, 16, 16) for i in range(3)]
    w3s = [_pad2(W3[:, 3 * f2 + i * f2:3 * f2 + (i + 1) * f2].T, 16, 16)
           for i in range(3)]
    b3p = _pad1(b3, 16)
    g3p = [_pad1(g3[i * f3:(i + 1) * f3], 16) for i in range(3)]
    be3p = [_pad1(be3[i * f3:(i + 1) * f3], 16) for i in range(3)]
    l1p = [_pad2(L1w[:, i * f3:(i + 1) * f3].T, 16, 32) for i in range(3)]
    l1bp = _pad1(L1b, 32)
    l2p = _pad2(L2w.T, 32, 16)
    l2bp = _pad1(L2b, 16)
    g4p = _pad1(g4, 16)
    be4p = _pad1(be4, 16)
    owp = _pad2(Ow.T, 16, 8)
    obp = _pad1(Ob, 8)

    hist = _sc_hist(edge_index)
    rdst, rsrc, meta = _sc_place(edge_index, hist)

    xd1, xs1 = _tc_call(functools.partial(_proj_body, fnext=f1),
                        (((N, 16)), ((N, 16))), x, wd1, ws1)
    ss1, sm1 = _sc_acc(rdst, rsrc, meta, xd1, xs1)

    xd2, xs2 = _tc_call(
        functools.partial(_mid_body, f=f1, fnext=f2),
        ((N, 16), (N, 16)),
        ss1[:N], sm1[:N], b1p, *g1p, *be1p, *w2d, *w2s)
    ss2, sm2 = _sc_acc(rdst, rsrc, meta, xd2, xs2)

    xd3, xs3 = _tc_call(
        functools.partial(_mid_body, f=f2, fnext=f3),
        ((N, 16), (N, 16)),
        ss2[:N], sm2[:N], b2p, *g2p, *be2p, *w3d, *w3s)
    ss3, sm3 = _sc_acc(rdst, rsrc, meta, xd3, xs3)

    (out,) = _tc_call(
        functools.partial(_head_body, f=f3),
        ((N, 1),),
        ss3[:N], sm3[:N], b3p, *g3p, *be3p,
        *l1p, l1bp, l2p, l2bp, g4p, be4p, owp, obp)
    return out


# speedup vs baseline: 4.0243x; 1.0003x over previous
import functools
import jax
import jax.numpy as jnp
from jax import lax
from jax.experimental import pallas as pl
from jax.experimental.pallas import tpu as pltpu, tpu_sc as plsc

N = 10000
E = 320000
D = 128
NT = 32
R = 320
NPAD = NT * R
EC = E // NT
EPAD = E + 1536
DUMMY = E + 1528
CH = 1024
NEG = -3.0e38

_MESH = dict(core_axis_name="c", subcore_axis_name="s", num_cores=2,
             num_subcores=16)

_i32 = jnp.int32
_f32 = jnp.float32


def _wid():
    return lax.axis_index("s") * 2 + lax.axis_index("c")


def _lane_iota():
    return lax.broadcasted_iota(_i32, (16,), 0)


def _bucket(d):
    return (d * 6554) >> 21


def _hist_body(edge_hbm, hist_hbm, dstv, histv, idxbuf):
    t = _wid()
    io = _lane_iota()
    pltpu.sync_copy(edge_hbm.at[1, pl.ds(t * EC, EC)], dstv.at[pl.ds(0, EC)])

    def zero16(i, _):
        histv[pl.ds(i * 16, 16)] = jnp.zeros((16,), _i32)
        return 0

    lax.fori_loop(0, 3, zero16, 0)

    def grp(g, _):
        d = dstv[pl.ds(g * 16, 16)]
        bs = plsc.sort_key_val(_bucket(d), io)[0]
        nxt = bs[jnp.minimum(io + 1, 15)]
        start = (bs != bs[jnp.maximum(io - 1, 0)]) | (io == 0)
        end = (bs != nxt) | (io == 15)
        rank = io - plsc.cummax(jnp.where(start, io, 0))
        plsc.addupdate_scatter(histv, [bs], rank + 1, mask=end)
        return 0

    lax.fori_loop(0, EC // 16, grp, 0)
    idxbuf[0, pl.ds(0, 16)] = io * 32 + t
    idxbuf[0, pl.ds(16, 16)] = (io + 16) * 32 + t
    pltpu.sync_copy(histv.at[pl.ds(0, 32)], hist_hbm.at[idxbuf.at[0]])


def _place_body(edge_hbm, hist_hbm, rdst_hbm, rsrc_hbm, meta_hbm,
                dstv, srcv, histv, cursor, metav, slots2d, sem):
    t = _wid()
    io = _lane_iota()
    pltpu.sync_copy(edge_hbm.at[0, pl.ds(t * EC, EC)], srcv.at[pl.ds(0, EC)])
    pltpu.sync_copy(edge_hbm.at[1, pl.ds(t * EC, EC)], dstv.at[pl.ds(0, EC)])
    pltpu.sync_copy(hist_hbm, histv.at[pl.ds(0, NT * 32)])

    def per_bucket(b, start):
        col0 = histv[pl.ds(b * 32, 16)]
        col1 = histv[pl.ds(b * 32 + 16, 16)]
        tot = jnp.sum(col0) + jnp.sum(col1)
        part = (jnp.sum(jnp.where(io < t, col0, 0))
                + jnp.sum(jnp.where(io + 16 < t, col1, 0)))
        b16 = b & ~15
        cw = cursor[pl.ds(b16, 16)]
        cursor[pl.ds(b16, 16)] = jnp.where(io == (b - b16), start + part, cw)
        mw = metav[pl.ds(b * 8, 16)]
        metav[pl.ds(b * 8, 16)] = jnp.where(io == 0, start, mw)
        mw2 = metav[pl.ds(256 + b * 8, 16)]
        metav[pl.ds(256 + b * 8, 16)] = jnp.where(io == 0, tot, mw2)
        return start + (((tot + 7) >> 3) << 3)

    with jax.named_scope("b2_offsets"):
        lax.fori_loop(0, 32, per_bucket, 0)

    @pl.when(t == 0)
    def _():
        pltpu.sync_copy(metav.at[pl.ds(0, 512)], meta_hbm)

    def initrow(i, _):
        slots2d[i >> 3, pl.ds((i & 7) * 16, 16)] = jnp.full((16,), DUMMY, _i32)
        return 0

    with jax.named_scope("b2_init"):
        lax.fori_loop(0, (EC + 240) // 16, initrow, 0)

    def grp(g, _):
        d = dstv[pl.ds(g * 16, 16)]
        s = srcv[pl.ds(g * 16, 16)]
        bs, ls = plsc.sort_key_val(_bucket(d), io)
        start = (bs != bs[jnp.maximum(io - 1, 0)]) | (io == 0)
        end = (bs != bs[jnp.minimum(io + 1, 15)]) | (io == 15)
        rank = io - plsc.cummax(jnp.where(start, io, 0))
        curs = plsc.load_gather(cursor, [bs])
        slots = curs + rank
        plsc.store_scatter(cursor, [bs], slots + 1, mask=end)
        row = jnp.full((16,), 0, _i32) + (g >> 3)
        col = (g & 7) * 16 + ls
        plsc.store_scatter(slots2d, [row, col], slots)
        return 0

    with jax.named_scope("b2_place"):
        lax.fori_loop(0, EC // 16, grp, 0)

    NJ = (EC + 240) // 128

    def scat8(blk, _):
        hs = []
        for u in range(4):
            j = blk * 4 + u
            hs.append(pltpu.async_copy(
                dstv.at[pl.ds(j * 128, 128)], rdst_hbm.at[slots2d.at[j]], sem))
            hs.append(pltpu.async_copy(
                srcv.at[pl.ds(j * 128, 128)], rsrc_hbm.at[slots2d.at[j]], sem))
        for h in hs:
            h.wait()
        return 0

    with jax.named_scope("b2_scat"):
        lax.fori_loop(0, NJ // 4, scat8, 0)


def _acc_body(rdst_hbm, rsrc_hbm, meta_hbm, xd_hbm, xs_hbm,
              ssum_hbm, smax_hbm,
              metav, dstg, srcg, dloc, rows_d, rows_s,
              acc_s, acc_m, acc_s1, acc_m1, acc_s2, acc_m2,
              acc_s3, acc_m3, sem):
    t = _wid()
    io = _lane_iota()
    pltpu.sync_copy(meta_hbm, metav.at[pl.ds(0, 512)])
    my_start = metav[pl.ds(t * 8, 16)][0]
    my_cnt = metav[pl.ds(256 + t * 8, 16)][0]

    def initacc(r, _):
        for s_ref, m_ref in ((acc_s, acc_m), (acc_s1, acc_m1),
                             (acc_s2, acc_m2), (acc_s3, acc_m3)):
            s_ref[r, pl.ds(0, 16)] = jnp.zeros((16,), _f32)
            m_ref[r, pl.ds(0, 16)] = jnp.full((16,), NEG, _f32)
        return 0

    lax.fori_loop(0, R + 8, initacc, 0)

    base_row = t * R
    sbank = (acc_s, acc_s1, acc_s2, acc_s3)
    mbank = (acc_m, acc_m1, acc_m2, acc_m3)

    def chunk(c, _):
        off = pl.multiple_of(my_start + c * CH, 8)
        pltpu.sync_copy(rdst_hbm.at[pl.ds(off, CH)], dstg)
        pltpu.sync_copy(rsrc_hbm.at[pl.ds(off, CH)], srcg)
        me = jnp.minimum(CH, my_cnt - c * CH)

        def mkidx(k, _):
            dg = dstg[pl.ds(k * 16, 16)]
            dg = jnp.minimum(jnp.maximum(dg, 0), N - 1)
            dstg[pl.ds(k * 16, 16)] = dg
            sg = srcg[pl.ds(k * 16, 16)]
            srcg[pl.ds(k * 16, 16)] = jnp.minimum(jnp.maximum(sg, 0), N - 1)
            dl = jnp.minimum(jnp.maximum(dg - base_row, 0), R)
            dloc[pl.ds(k * 16, 16)] = jnp.where(k * 16 + io < me, dl, R)
            return 0

        lax.fori_loop(0, CH // 16, mkidx, 0)

        handles = []
        for k in range(CH // 128):
            handles.append(pltpu.async_copy(
                xd_hbm.at[dstg.at[pl.ds(k * 128, 128)]],
                rows_d.at[pl.ds(k * 128, 128), :], sem))
            handles.append(pltpu.async_copy(
                xs_hbm.at[srcg.at[pl.ds(k * 128, 128)]],
                rows_s.at[pl.ds(k * 128, 128), :], sem))
        for h in handles:
            h.wait()

        def grp16(g, _):
            dlv = dloc[pl.ds(g * 16, 16)]
            for k in range(16):
                e = g * 16 + k
                dl = dlv[k]
                v = rows_d[e] + rows_s[e]
                s_ref = sbank[k & 3]
                m_ref = mbank[k & 3]
                s_ref[dl, pl.ds(0, 16)] = s_ref[dl, pl.ds(0, 16)] + v
                m_ref[dl, pl.ds(0, 16)] = jnp.maximum(m_ref[dl, pl.ds(0, 16)], v)
            return 0

        lax.fori_loop(0, (me + 15) >> 4, grp16, 0)
        return 0

    nchunks = (my_cnt + CH - 1) >> 10
    lax.fori_loop(0, nchunks, chunk, 0)

    def mrg(r, _):
        rs = pl.ds(0, 16)
        acc_s[r, rs] = (acc_s[r, rs] + acc_s1[r, rs]) + (acc_s2[r, rs] + acc_s3[r, rs])
        acc_m[r, rs] = jnp.maximum(jnp.maximum(acc_m[r, rs], acc_m1[r, rs]),
                                   jnp.maximum(acc_m2[r, rs], acc_m3[r, rs]))
        return 0

    lax.fori_loop(0, R, mrg, 0)

    pltpu.sync_copy(acc_s.at[pl.ds(0, R), :], ssum_hbm.at[pl.ds(base_row, R), :])
    pltpu.sync_copy(acc_m.at[pl.ds(0, R), :], smax_hbm.at[pl.ds(base_row, R), :])


def _lrelu(t):
    return jnp.where(t >= 0, t, 0.01 * t)


def _proj_body(x_ref, wd_ref, ws_ref, od_ref, os_ref, *, fnext):
    cc = 0.5 * (lax.broadcasted_iota(_i32, (1, 16), 1) == fnext).astype(_f32)
    x = x_ref[...]
    od_ref[...] = jax.lax.dot_general(
        x, wd_ref[...], (((1,), (0,)), ((), ())),
        preferred_element_type=_f32) + cc
    os_ref[...] = jax.lax.dot_general(
        x, ws_ref[...], (((1,), (0,)), ((), ())),
        preferred_element_type=_f32) + cc


def _bn_blocks(ssum, smax, bias, f, gs, bes):
    cnt = ssum[:, f:f + 1]
    s_sum = ssum + cnt * bias
    s_mean = s_sum / jnp.maximum(cnt, 1.0)
    s_max = jnp.where(cnt > 0, smax + bias, 0.0)
    outs = []
    for h, g, be in zip((s_sum, s_mean, s_max), gs, bes):
        mu = jnp.mean(h, axis=0, keepdims=True)
        var = jnp.mean((h - mu) ** 2, axis=0, keepdims=True)
        hn = (h - mu) * lax.rsqrt(var + 1e-5) * g + be
        outs.append(_lrelu(hn))
    return outs


def _mid_body(ssum_ref, smax_ref, b_ref, g1_ref, g2_ref, g3_ref,
              be1_ref, be2_ref, be3_ref,
              da_ref, db_ref, dc_ref, sa_ref, sb_ref, sc_ref,
              od_ref, os_ref, *, f, fnext):
    hs, hm, hx = _bn_blocks(ssum_ref[...], smax_ref[...], b_ref[...], f,
                            (g1_ref[...], g2_ref[...], g3_ref[...]),
                            (be1_ref[...], be2_ref[...], be3_ref[...]))
    cc = 0.5 * (lax.broadcasted_iota(_i32, (1, 16), 1) == fnext).astype(_f32)
    dn = (((1,), (0,)), ((), ()))
    od_ref[...] = (jax.lax.dot_general(hs, da_ref[...], dn, preferred_element_type=_f32)
                   + jax.lax.dot_general(hm, db_ref[...], dn, preferred_element_type=_f32)
                   + jax.lax.dot_general(hx, dc_ref[...], dn, preferred_element_type=_f32)
                   + cc)
    os_ref[...] = (jax.lax.dot_general(hs, sa_ref[...], dn, preferred_element_type=_f32)
                   + jax.lax.dot_general(hm, sb_ref[...], dn, preferred_element_type=_f32)
                   + jax.lax.dot_general(hx, sc_ref[...], dn, preferred_element_type=_f32)
                   + cc)


def _head_body(ssum_ref, smax_ref, b_ref, g1_ref, g2_ref, g3_ref,
               be1_ref, be2_ref, be3_ref,
               la_ref, lb_ref, lc_ref, l1b_ref, l2_ref, l2b_ref,
               g4_ref, be4_ref, ow_ref, ob_ref, out_ref, *, f):
    hs, hm, hx = _bn_blocks(ssum_ref[...], smax_ref[...], b_ref[...], f,
                            (g1_ref[...], g2_ref[...], g3_ref[...]),
                            (be1_ref[...], be2_ref[...], be3_ref[...]))
    dn = (((1,), (0,)), ((), ()))
    v = _lrelu(jax.lax.dot_general(hs, la_ref[...], dn, preferred_element_type=_f32)
               + jax.lax.dot_general(hm, lb_ref[...], dn, preferred_element_type=_f32)
               + jax.lax.dot_general(hx, lc_ref[...], dn, preferred_element_type=_f32)
               + l1b_ref[...])
    z = jax.lax.dot_general(v, l2_ref[...], dn, preferred_element_type=_f32) + l2b_ref[...]
    mu = jnp.mean(z, axis=0, keepdims=True)
    var = jnp.mean((z - mu) ** 2, axis=0, keepdims=True)
    z = _lrelu((z - mu) * lax.rsqrt(var + 1e-5) * g4_ref[...] + be4_ref[...])
    res = jax.lax.dot_general(z, ow_ref[...], dn, preferred_element_type=_f32) + ob_ref[...]
    out_ref[...] = res[:, 0:1]


def _pad2(a, rows, cols):
    return jnp.pad(a, ((0, rows - a.shape[0]), (0, cols - a.shape[1])))


def _pad1(a, n):
    return jnp.pad(a, (0, n - a.shape[0])).reshape(1, n)


def _tc_call(body, out_shapes, *args):
    return pl.pallas_call(
        body,
        out_shape=tuple(jax.ShapeDtypeStruct(s, _f32) for s in out_shapes),
    )(*args)


def _sc_hist(edge_index):
    return pl.kernel(
        _hist_body,
        out_type=jax.ShapeDtypeStruct((NT * 32,), _i32),
        mesh=plsc.VectorSubcoreMesh(**_MESH),
        compiler_params=pltpu.CompilerParams(use_tc_tiling_on_sc=False,
                                             needs_layout_passes=False),
        scratch_types=[
            pltpu.VMEM((EC + 16,), _i32),
            pltpu.VMEM((1040,), _i32),
            pltpu.VMEM((1, 32), _i32),
        ],
    )(edge_index)


def _sc_place(edge_index, hist):
    return pl.kernel(
        _place_body,
        out_type=(jax.ShapeDtypeStruct((EPAD,), _i32),
                  jax.ShapeDtypeStruct((EPAD,), _i32),
                  jax.ShapeDtypeStruct((512,), _i32)),
        mesh=plsc.VectorSubcoreMesh(**_MESH),
        compiler_params=pltpu.CompilerParams(use_tc_tiling_on_sc=False,
                                             needs_layout_passes=False),
        scratch_types=[
            pltpu.VMEM((EC + 240,), _i32),
            pltpu.VMEM((EC + 240,), _i32),
            pltpu.VMEM((NT * 32 + 16,), _i32),
            pltpu.VMEM((48,), _i32),
            pltpu.VMEM((528,), _i32),
            pltpu.VMEM(((EC + 240) // 128, 128), _i32),
            pltpu.SemaphoreType.DMA,
        ],
    )(edge_index, hist)


def _sc_acc(rdst, rsrc, meta, xd, xs):
    return pl.kernel(
        _acc_body,
        out_type=(jax.ShapeDtypeStruct((NPAD, 16), _f32),
                  jax.ShapeDtypeStruct((NPAD, 16), _f32)),
        mesh=plsc.VectorSubcoreMesh(**_MESH),
        compiler_params=pltpu.CompilerParams(use_tc_tiling_on_sc=False),
        scratch_types=[
            pltpu.VMEM((528,), _i32),
            pltpu.VMEM((CH,), _i32),
            pltpu.VMEM((CH,), _i32),
            pltpu.VMEM((CH,), _i32),
            pltpu.VMEM((CH, 16), _f32),
            pltpu.VMEM((CH, 16), _f32),
            pltpu.VMEM((R + 8, 16), _f32),
            pltpu.VMEM((R + 8, 16), _f32),
            pltpu.VMEM((R + 8, 16), _f32),
            pltpu.VMEM((R + 8, 16), _f32),
            pltpu.VMEM((R + 8, 16), _f32),
            pltpu.VMEM((R + 8, 16), _f32),
            pltpu.VMEM((R + 8, 16), _f32),
            pltpu.VMEM((R + 8, 16), _f32),
            pltpu.SemaphoreType.DMA,
        ],
    )(rdst, rsrc, meta, xd, xs)


def kernel(x, edge_index, W1, b1, g1, be1, W2, b2, g2, be2, W3, b3, g3, be3,
           L1w, L1b, L2w, L2b, g4, be4, Ow, Ob):
    f1, f2, f3 = 12, 9, 7

    wd1 = _pad2(W1[:, :D].T, D, 16)
    ws1 = _pad2(W1[:, D:].T, D, 16)
    b1p = _pad1(b1, 16)
    g1p = [_pad1(g1[i * f1:(i + 1) * f1], 16) for i in range(3)]
    be1p = [_pad1(be1[i * f1:(i + 1) * f1], 16) for i in range(3)]
    w2d = [_pad2(W2[:, i * f1:(i + 1) * f1].T, 16, 16) for i in range(3)]
    w2s = [_pad2(W2[:, 3 * f1 + i * f1:3 * f1 + (i + 1) * f1].T, 16, 16)
           for i in range(3)]
    b2p = _pad1(b2, 16)
    g2p = [_pad1(g2[i * f2:(i + 1) * f2], 16) for i in range(3)]
    be2p = [_pad1(be2[i * f2:(i + 1) * f2], 16) for i in range(3)]
    w3d = [_pad2(W3[:, i * f2:(i + 1) * f2].T, 16, 16) for i in range(3)]
    w3s = [_pad2(W3[:, 3 * f2 + i * f2:3 * f2 + (i + 1) * f2].T, 16, 16)
           for i in range(3)]
    b3p = _pad1(b3, 16)
    g3p = [_pad1(g3[i * f3:(i + 1) * f3], 16) for i in range(3)]
    be3p = [_pad1(be3[i * f3:(i + 1) * f3], 16) for i in range(3)]
    l1p = [_pad2(L1w[:, i * f3:(i + 1) * f3].T, 16, 32) for i in range(3)]
    l1bp = _pad1(L1b, 32)
    l2p = _pad2(L2w.T, 32, 16)
    l2bp = _pad1(L2b, 16)
    g4p = _pad1(g4, 16)
    be4p = _pad1(be4, 16)
    owp = _pad2(Ow.T, 16, 8)
    obp = _pad1(Ob, 8)

    hist = _sc_hist(edge_index)
    rdst, rsrc, meta = _sc_place(edge_index, hist)

    xd1, xs1 = _tc_call(functools.partial(_proj_body, fnext=f1),
                        (((N, 16)), ((N, 16))), x, wd1, ws1)
    ss1, sm1 = _sc_acc(rdst, rsrc, meta, xd1, xs1)

    xd2, xs2 = _tc_call(
        functools.partial(_mid_body, f=f1, fnext=f2),
        ((N, 16), (N, 16)),
        ss1[:N], sm1[:N], b1p, *g1p, *be1p, *w2d, *w2s)
    ss2, sm2 = _sc_acc(rdst, rsrc, meta, xd2, xs2)

    xd3, xs3 = _tc_call(
        functools.partial(_mid_body, f=f2, fnext=f3),
        ((N, 16), (N, 16)),
        ss2[:N], sm2[:N], b2p, *g2p, *be2p, *w3d, *w3s)
    ss3, sm3 = _sc_acc(rdst, rsrc, meta, xd3, xs3)

    (out,) = _tc_call(
        functools.partial(_head_body, f=f3),
        ((N, 1),),
        ss3[:N], sm3[:N], b3p, *g3p, *be3p,
        *l1p, l1bp, l2p, l2bp, g4p, be4p, owp, obp)
    return out


# speedup vs baseline: 14.8194x; 3.6825x over previous
import functools
import jax
import jax.numpy as jnp
from jax import lax
from jax.experimental import pallas as pl
from jax.experimental.pallas import tpu as pltpu, tpu_sc as plsc

N = 10000
E = 320000
D = 128
NT = 32
R = 320
NPAD = NT * R
EC = E // NT
EPAD = E + 8192
SENT = 1 << 20
CH = 1024
NEG = -3.0e38

_MESH = dict(core_axis_name="c", subcore_axis_name="s", num_cores=2,
             num_subcores=16)

_i32 = jnp.int32
_f32 = jnp.float32


def _wid():
    return lax.axis_index("s") * 2 + lax.axis_index("c")


def _lane_iota():
    return lax.broadcasted_iota(_i32, (16,), 0)


def _bucket(d):
    return (d * 6554) >> 21


def _hist_body(edge_hbm, hist_hbm, dstv, histv, idxbuf):
    t = _wid()
    io = _lane_iota()
    pltpu.sync_copy(edge_hbm.at[1, pl.ds(t * EC, EC)], dstv.at[pl.ds(0, EC)])

    def zero16(i, _):
        histv[pl.ds(i * 16, 16)] = jnp.zeros((16,), _i32)
        return 0

    lax.fori_loop(0, 3, zero16, 0)

    def grp(g, _):
        d = dstv[pl.ds(g * 16, 16)]
        bs = plsc.sort_key_val(_bucket(d), io)[0]
        nxt = bs[jnp.minimum(io + 1, 15)]
        start = (bs != bs[jnp.maximum(io - 1, 0)]) | (io == 0)
        end = (bs != nxt) | (io == 15)
        rank = io - plsc.cummax(jnp.where(start, io, 0))
        plsc.addupdate_scatter(histv, [bs], rank + 1, mask=end)
        return 0

    lax.fori_loop(0, EC // 16, grp, 0)
    idxbuf[0, pl.ds(0, 16)] = io * 32 + t
    idxbuf[0, pl.ds(16, 16)] = (io + 16) * 32 + t
    pltpu.sync_copy(histv.at[pl.ds(0, 32)], hist_hbm.at[idxbuf.at[0]])


def _place_body(edge_hbm, hist_hbm, rdst_hbm, rsrc_hbm, meta_hbm,
                dstv, srcv, histv, cursor, metav, lmeta, ldst, lsrc):
    t = _wid()
    io = _lane_iota()
    pltpu.sync_copy(edge_hbm.at[0, pl.ds(t * EC, EC)], srcv.at[pl.ds(0, EC)])
    pltpu.sync_copy(edge_hbm.at[1, pl.ds(t * EC, EC)], dstv.at[pl.ds(0, EC)])
    pltpu.sync_copy(hist_hbm, histv.at[pl.ds(0, NT * 32)])

    def per_bucket(b, carry):
        gstart, lstart = carry
        col0 = histv[pl.ds(b * 32, 16)]
        col1 = histv[pl.ds(b * 32 + 16, 16)]
        r0 = ((col0 + 7) >> 3) << 3
        r1 = ((col1 + 7) >> 3) << 3
        tot8 = jnp.sum(r0) + jnp.sum(r1)
        part8 = (jnp.sum(jnp.where(io < t, r0, 0))
                 + jnp.sum(jnp.where(io + 16 < t, r1, 0)))
        my8 = (jnp.sum(jnp.where(io == t, r0, 0))
               + jnp.sum(jnp.where(io + 16 == t, r1, 0)))
        b16 = b & ~15
        cw = cursor[pl.ds(b16, 16)]
        cursor[pl.ds(b16, 16)] = jnp.where(io == (b - b16), lstart, cw)
        w = lmeta[pl.ds(b * 8, 16)]
        lmeta[pl.ds(b * 8, 16)] = jnp.where(io == 0, lstart, w)
        w = lmeta[pl.ds(256 + b * 8, 16)]
        lmeta[pl.ds(256 + b * 8, 16)] = jnp.where(io == 0, my8, w)
        w = lmeta[pl.ds(512 + b * 8, 16)]
        lmeta[pl.ds(512 + b * 8, 16)] = jnp.where(io == 0, gstart + part8, w)
        w = metav[pl.ds(b * 8, 16)]
        metav[pl.ds(b * 8, 16)] = jnp.where(io == 0, gstart, w)
        w = metav[pl.ds(256 + b * 8, 16)]
        metav[pl.ds(256 + b * 8, 16)] = jnp.where(io == 0, tot8, w)
        return (gstart + tot8, lstart + my8)

    with jax.named_scope("b2_offsets"):
        lax.fori_loop(0, 32, per_bucket, (0, 0))

    @pl.when(t == 0)
    def _():
        pltpu.sync_copy(metav.at[pl.ds(0, 512)], meta_hbm)

    def initsent(i, _):
        ldst[pl.ds(i * 16, 16)] = jnp.full((16,), SENT, _i32)
        lsrc[pl.ds(i * 16, 16)] = jnp.zeros((16,), _i32)
        return 0

    with jax.named_scope("b2_init"):
        lax.fori_loop(0, (EC + 256) // 16, initsent, 0)

    def grp(g, _):
        d = dstv[pl.ds(g * 16, 16)]
        s = srcv[pl.ds(g * 16, 16)]
        bs, ls = plsc.sort_key_val(_bucket(d), io)
        start = (bs != bs[jnp.maximum(io - 1, 0)]) | (io == 0)
        end = (bs != bs[jnp.minimum(io + 1, 15)]) | (io == 15)
        rank = io - plsc.cummax(jnp.where(start, io, 0))
        curs = plsc.load_gather(cursor, [bs])
        slots = curs + rank
        plsc.store_scatter(cursor, [bs], slots + 1, mask=end)
        plsc.store_scatter(ldst, [slots], d[ls])
        plsc.store_scatter(lsrc, [slots], s[ls])
        return 0

    with jax.named_scope("b2_place"):
        lax.fori_loop(0, EC // 16, grp, 0)

    def wout(b, _):
        lo = pl.multiple_of(lmeta[pl.ds(b * 8, 16)][0], 8)
        ln = lmeta[pl.ds(256 + b * 8, 16)][0]
        go = pl.multiple_of(lmeta[pl.ds(512 + b * 8, 16)][0], 8)

        def w64(k, _):
            so = pl.multiple_of(lo + k * 64, 8)
            do = pl.multiple_of(go + k * 64, 8)
            pltpu.sync_copy(ldst.at[pl.ds(so, 64)], rdst_hbm.at[pl.ds(do, 64)])
            pltpu.sync_copy(lsrc.at[pl.ds(so, 64)], rsrc_hbm.at[pl.ds(do, 64)])
            return 0

        lax.fori_loop(0, ln >> 6, w64, 0)
        full = (ln >> 6) << 6

        def w8(k, _):
            so = pl.multiple_of(lo + full + k * 8, 8)
            do = pl.multiple_of(go + full + k * 8, 8)
            pltpu.sync_copy(ldst.at[pl.ds(so, 8)], rdst_hbm.at[pl.ds(do, 8)])
            pltpu.sync_copy(lsrc.at[pl.ds(so, 8)], rsrc_hbm.at[pl.ds(do, 8)])
            return 0

        lax.fori_loop(0, (ln & 63) >> 3, w8, 0)
        return 0

    with jax.named_scope("b2_scat"):
        lax.fori_loop(0, 32, wout, 0)


def _acc_body(rdst_hbm, rsrc_hbm, meta_hbm, xd_hbm, xs_hbm,
              ssum_hbm, smax_hbm,
              metav, dstg, srcg, dloc, rows_d, rows_s,
              acc_s, acc_m, acc_s1, acc_m1, acc_s2, acc_m2,
              acc_s3, acc_m3, sem):
    t = _wid()
    io = _lane_iota()
    pltpu.sync_copy(meta_hbm, metav.at[pl.ds(0, 512)])
    my_start = metav[pl.ds(t * 8, 16)][0]
    my_cnt = metav[pl.ds(256 + t * 8, 16)][0]

    def initacc(r, _):
        for s_ref, m_ref in ((acc_s, acc_m), (acc_s1, acc_m1),
                             (acc_s2, acc_m2), (acc_s3, acc_m3)):
            s_ref[r, pl.ds(0, 16)] = jnp.zeros((16,), _f32)
            m_ref[r, pl.ds(0, 16)] = jnp.full((16,), NEG, _f32)
        return 0

    lax.fori_loop(0, R + 8, initacc, 0)

    base_row = t * R
    sbank = (acc_s, acc_s1, acc_s2, acc_s3)
    mbank = (acc_m, acc_m1, acc_m2, acc_m3)

    def chunk(c, _):
        off = pl.multiple_of(my_start + c * CH, 8)
        pltpu.sync_copy(rdst_hbm.at[pl.ds(off, CH)], dstg)
        pltpu.sync_copy(rsrc_hbm.at[pl.ds(off, CH)], srcg)
        me = jnp.minimum(CH, my_cnt - c * CH)

        def mkidx(k, _):
            dg = dstg[pl.ds(k * 16, 16)]
            dl = jnp.minimum(jnp.maximum(dg - base_row, 0), R)
            dstg[pl.ds(k * 16, 16)] = jnp.minimum(jnp.maximum(dg, 0), N - 1)
            sg = srcg[pl.ds(k * 16, 16)]
            srcg[pl.ds(k * 16, 16)] = jnp.minimum(jnp.maximum(sg, 0), N - 1)
            dloc[pl.ds(k * 16, 16)] = jnp.where(k * 16 + io < me, dl, R)
            return 0

        lax.fori_loop(0, CH // 16, mkidx, 0)

        handles = []
        for k in range(CH // 128):
            handles.append(pltpu.async_copy(
                xd_hbm.at[dstg.at[pl.ds(k * 128, 128)]],
                rows_d.at[pl.ds(k * 128, 128), :], sem))
            handles.append(pltpu.async_copy(
                xs_hbm.at[srcg.at[pl.ds(k * 128, 128)]],
                rows_s.at[pl.ds(k * 128, 128), :], sem))
        for h in handles:
            h.wait()

        def grp16(g, _):
            dlv = dloc[pl.ds(g * 16, 16)]
            for k in range(16):
                e = g * 16 + k
                dl = dlv[k]
                v = rows_d[e] + rows_s[e]
                s_ref = sbank[k & 3]
                m_ref = mbank[k & 3]
                s_ref[dl, pl.ds(0, 16)] = s_ref[dl, pl.ds(0, 16)] + v
                m_ref[dl, pl.ds(0, 16)] = jnp.maximum(m_ref[dl, pl.ds(0, 16)], v)
            return 0

        lax.fori_loop(0, (me + 15) >> 4, grp16, 0)
        return 0

    nchunks = (my_cnt + CH - 1) >> 10
    lax.fori_loop(0, nchunks, chunk, 0)

    def mrg(r, _):
        rs = pl.ds(0, 16)
        acc_s[r, rs] = (acc_s[r, rs] + acc_s1[r, rs]) + (acc_s2[r, rs] + acc_s3[r, rs])
        acc_m[r, rs] = jnp.maximum(jnp.maximum(acc_m[r, rs], acc_m1[r, rs]),
                                   jnp.maximum(acc_m2[r, rs], acc_m3[r, rs]))
        return 0

    lax.fori_loop(0, R, mrg, 0)

    pltpu.sync_copy(acc_s.at[pl.ds(0, R), :], ssum_hbm.at[pl.ds(base_row, R), :])
    pltpu.sync_copy(acc_m.at[pl.ds(0, R), :], smax_hbm.at[pl.ds(base_row, R), :])


def _lrelu(t):
    return jnp.where(t >= 0, t, 0.01 * t)


def _proj_body(x_ref, wd_ref, ws_ref, od_ref, os_ref, *, fnext):
    cc = 0.5 * (lax.broadcasted_iota(_i32, (1, 16), 1) == fnext).astype(_f32)
    x = x_ref[...]
    od_ref[...] = jax.lax.dot_general(
        x, wd_ref[...], (((1,), (0,)), ((), ())),
        preferred_element_type=_f32) + cc
    os_ref[...] = jax.lax.dot_general(
        x, ws_ref[...], (((1,), (0,)), ((), ())),
        preferred_element_type=_f32) + cc


def _bn_blocks(ssum, smax, bias, f, gs, bes):
    cnt = ssum[:, f:f + 1]
    s_sum = ssum + cnt * bias
    s_mean = s_sum / jnp.maximum(cnt, 1.0)
    s_max = jnp.where(cnt > 0, smax + bias, 0.0)
    outs = []
    for h, g, be in zip((s_sum, s_mean, s_max), gs, bes):
        mu = jnp.mean(h, axis=0, keepdims=True)
        var = jnp.mean((h - mu) ** 2, axis=0, keepdims=True)
        hn = (h - mu) * lax.rsqrt(var + 1e-5) * g + be
        outs.append(_lrelu(hn))
    return outs


def _mid_body(ssum_ref, smax_ref, b_ref, g1_ref, g2_ref, g3_ref,
              be1_ref, be2_ref, be3_ref,
              da_ref, db_ref, dc_ref, sa_ref, sb_ref, sc_ref,
              od_ref, os_ref, *, f, fnext):
    hs, hm, hx = _bn_blocks(ssum_ref[...], smax_ref[...], b_ref[...], f,
                            (g1_ref[...], g2_ref[...], g3_ref[...]),
                            (be1_ref[...], be2_ref[...], be3_ref[...]))
    cc = 0.5 * (lax.broadcasted_iota(_i32, (1, 16), 1) == fnext).astype(_f32)
    dn = (((1,), (0,)), ((), ()))
    od_ref[...] = (jax.lax.dot_general(hs, da_ref[...], dn, preferred_element_type=_f32)
                   + jax.lax.dot_general(hm, db_ref[...], dn, preferred_element_type=_f32)
                   + jax.lax.dot_general(hx, dc_ref[...], dn, preferred_element_type=_f32)
                   + cc)
    os_ref[...] = (jax.lax.dot_general(hs, sa_ref[...], dn, preferred_element_type=_f32)
                   + jax.lax.dot_general(hm, sb_ref[...], dn, preferred_element_type=_f32)
                   + jax.lax.dot_general(hx, sc_ref[...], dn, preferred_element_type=_f32)
                   + cc)


def _head_body(ssum_ref, smax_ref, b_ref, g1_ref, g2_ref, g3_ref,
               be1_ref, be2_ref, be3_ref,
               la_ref, lb_ref, lc_ref, l1b_ref, l2_ref, l2b_ref,
               g4_ref, be4_ref, ow_ref, ob_ref, out_ref, *, f):
    hs, hm, hx = _bn_blocks(ssum_ref[...], smax_ref[...], b_ref[...], f,
                            (g1_ref[...], g2_ref[...], g3_ref[...]),
                            (be1_ref[...], be2_ref[...], be3_ref[...]))
    dn = (((1,), (0,)), ((), ()))
    v = _lrelu(jax.lax.dot_general(hs, la_ref[...], dn, preferred_element_type=_f32)
               + jax.lax.dot_general(hm, lb_ref[...], dn, preferred_element_type=_f32)
               + jax.lax.dot_general(hx, lc_ref[...], dn, preferred_element_type=_f32)
               + l1b_ref[...])
    z = jax.lax.dot_general(v, l2_ref[...], dn, preferred_element_type=_f32) + l2b_ref[...]
    mu = jnp.mean(z, axis=0, keepdims=True)
    var = jnp.mean((z - mu) ** 2, axis=0, keepdims=True)
    z = _lrelu((z - mu) * lax.rsqrt(var + 1e-5) * g4_ref[...] + be4_ref[...])
    res = jax.lax.dot_general(z, ow_ref[...], dn, preferred_element_type=_f32) + ob_ref[...]
    out_ref[...] = res[:, 0:1]


def _pad2(a, rows, cols):
    return jnp.pad(a, ((0, rows - a.shape[0]), (0, cols - a.shape[1])))


def _pad1(a, n):
    return jnp.pad(a, (0, n - a.shape[0])).reshape(1, n)


def _tc_call(body, out_shapes, *args):
    return pl.pallas_call(
        body,
        out_shape=tuple(jax.ShapeDtypeStruct(s, _f32) for s in out_shapes),
    )(*args)


def _sc_hist(edge_index):
    return pl.kernel(
        _hist_body,
        out_type=jax.ShapeDtypeStruct((NT * 32,), _i32),
        mesh=plsc.VectorSubcoreMesh(**_MESH),
        compiler_params=pltpu.CompilerParams(use_tc_tiling_on_sc=False,
                                             needs_layout_passes=False),
        scratch_types=[
            pltpu.VMEM((EC + 16,), _i32),
            pltpu.VMEM((1040,), _i32),
            pltpu.VMEM((1, 32), _i32),
        ],
    )(edge_index)


def _sc_place(edge_index, hist):
    return pl.kernel(
        _place_body,
        out_type=(jax.ShapeDtypeStruct((EPAD,), _i32),
                  jax.ShapeDtypeStruct((EPAD,), _i32),
                  jax.ShapeDtypeStruct((512,), _i32)),
        mesh=plsc.VectorSubcoreMesh(**_MESH),
        compiler_params=pltpu.CompilerParams(use_tc_tiling_on_sc=False,
                                             needs_layout_passes=False),
        scratch_types=[
            pltpu.VMEM((EC,), _i32),
            pltpu.VMEM((EC,), _i32),
            pltpu.VMEM((NT * 32 + 16,), _i32),
            pltpu.VMEM((48,), _i32),
            pltpu.VMEM((528,), _i32),
            pltpu.VMEM((784,), _i32),
            pltpu.VMEM((EC + 256,), _i32),
            pltpu.VMEM((EC + 256,), _i32),
        ],
    )(edge_index, hist)


def _sc_acc(rdst, rsrc, meta, xd, xs):
    return pl.kernel(
        _acc_body,
        out_type=(jax.ShapeDtypeStruct((NPAD, 16), _f32),
                  jax.ShapeDtypeStruct((NPAD, 16), _f32)),
        mesh=plsc.VectorSubcoreMesh(**_MESH),
        compiler_params=pltpu.CompilerParams(use_tc_tiling_on_sc=False),
        scratch_types=[
            pltpu.VMEM((528,), _i32),
            pltpu.VMEM((CH,), _i32),
            pltpu.VMEM((CH,), _i32),
            pltpu.VMEM((CH,), _i32),
            pltpu.VMEM((CH, 16), _f32),
            pltpu.VMEM((CH, 16), _f32),
            pltpu.VMEM((R + 8, 16), _f32),
            pltpu.VMEM((R + 8, 16), _f32),
            pltpu.VMEM((R + 8, 16), _f32),
            pltpu.VMEM((R + 8, 16), _f32),
            pltpu.VMEM((R + 8, 16), _f32),
            pltpu.VMEM((R + 8, 16), _f32),
            pltpu.VMEM((R + 8, 16), _f32),
            pltpu.VMEM((R + 8, 16), _f32),
            pltpu.SemaphoreType.DMA,
        ],
    )(rdst, rsrc, meta, xd, xs)


def kernel(x, edge_index, W1, b1, g1, be1, W2, b2, g2, be2, W3, b3, g3, be3,
           L1w, L1b, L2w, L2b, g4, be4, Ow, Ob):
    f1, f2, f3 = 12, 9, 7

    wd1 = _pad2(W1[:, :D].T, D, 16)
    ws1 = _pad2(W1[:, D:].T, D, 16)
    b1p = _pad1(b1, 16)
    g1p = [_pad1(g1[i * f1:(i + 1) * f1], 16) for i in range(3)]
    be1p = [_pad1(be1[i * f1:(i + 1) * f1], 16) for i in range(3)]
    w2d = [_pad2(W2[:, i * f1:(i + 1) * f1].T, 16, 16) for i in range(3)]
    w2s = [_pad2(W2[:, 3 * f1 + i * f1:3 * f1 + (i + 1) * f1].T, 16, 16)
           for i in range(3)]
    b2p = _pad1(b2, 16)
    g2p = [_pad1(g2[i * f2:(i + 1) * f2], 16) for i in range(3)]
    be2p = [_pad1(be2[i * f2:(i + 1) * f2], 16) for i in range(3)]
    w3d = [_pad2(W3[:, i * f2:(i + 1) * f2].T, 16, 16) for i in range(3)]
    w3s = [_pad2(W3[:, 3 * f2 + i * f2:3 * f2 + (i + 1) * f2].T, 16, 16)
           for i in range(3)]
    b3p = _pad1(b3, 16)
    g3p = [_pad1(g3[i * f3:(i + 1) * f3], 16) for i in range(3)]
    be3p = [_pad1(be3[i * f3:(i + 1) * f3], 16) for i in range(3)]
    l1p = [_pad2(L1w[:, i * f3:(i + 1) * f3].T, 16, 32) for i in range(3)]
    l1bp = _pad1(L1b, 32)
    l2p = _pad2(L2w.T, 32, 16)
    l2bp = _pad1(L2b, 16)
    g4p = _pad1(g4, 16)
    be4p = _pad1(be4, 16)
    owp = _pad2(Ow.T, 16, 8)
    obp = _pad1(Ob, 8)

    hist = _sc_hist(edge_index)
    rdst, rsrc, meta = _sc_place(edge_index, hist)

    xd1, xs1 = _tc_call(functools.partial(_proj_body, fnext=f1),
                        (((N, 16)), ((N, 16))), x, wd1, ws1)
    ss1, sm1 = _sc_acc(rdst, rsrc, meta, xd1, xs1)

    xd2, xs2 = _tc_call(
        functools.partial(_mid_body, f=f1, fnext=f2),
        ((N, 16), (N, 16)),
        ss1[:N], sm1[:N], b1p, *g1p, *be1p, *w2d, *w2s)
    ss2, sm2 = _sc_acc(rdst, rsrc, meta, xd2, xs2)

    xd3, xs3 = _tc_call(
        functools.partial(_mid_body, f=f2, fnext=f3),
        ((N, 16), (N, 16)),
        ss2[:N], sm2[:N], b2p, *g2p, *be2p, *w3d, *w3s)
    ss3, sm3 = _sc_acc(rdst, rsrc, meta, xd3, xs3)

    (out,) = _tc_call(
        functools.partial(_head_body, f=f3),
        ((N, 1),),
        ss3[:N], sm3[:N], b3p, *g3p, *be3p,
        *l1p, l1bp, l2p, l2bp, g4p, be4p, owp, obp)
    return out
